# baseline (speedup 1.0000x reference)
import functools

import jax
import jax.numpy as jnp
from jax import lax
from jax.experimental import pallas as pl
from jax.experimental.pallas import tpu as pltpu
from jax.experimental.pallas import tpu_sc as plsc

N = 10000
D = 128
E = 320000
ET = E + N
K = 128
NC = 2
NS = 16
NW = NC * NS
NB = -(-ET // (NW * K))
NB0 = 115
NB1 = 47
CH = 40
NCH = 3
WIN = CH + 8
_max_start = max(NB * (NW - 1), NS * NB0 + (NS - 1) * NB1)
NBLK = (_max_start // 8 * 8 + (NCH - 1) * CH + WIN + 7) // 8 * 8
EPAD = NBLK * K
NPAD = 10240
RPT = NPAD // NS
EPS = 1e-5

_mesh = plsc.VectorSubcoreMesh(
    core_axis_name="c", subcore_axis_name="s", num_cores=NC, num_subcores=NS)


@functools.partial(
    pl.kernel,
    out_type=jax.ShapeDtypeStruct((NC, NPAD, D), jnp.float32),
    mesh=_mesh,
    scratch_types=[
        pltpu.VMEM((NB + 7, K), jnp.int32),
        pltpu.VMEM((K, D), jnp.float32),
        pltpu.VMEM((16, D), jnp.float32),
        pltpu.VMEM_SHARED((NPAD, D), jnp.float32),
    ],
)
def _sc_deg(e2_hbm, degp_hbm, didx, ones_v, zbuf, degacc):
    cid = lax.axis_index("c")
    sid = lax.axis_index("s")
    w = cid * NS + sid
    b0 = pl.multiple_of(NB * w // 8 * 8, 8)
    off = NB * w - b0

    def fill_ones(i, c):
        for cc in range(D // 16):
            ones_v[i, pl.ds(cc * 16, 16)] = jnp.ones((16,), jnp.float32)
        return c
    lax.fori_loop(0, K, fill_ones, 0)

    def fill_zero(i, c):
        for cc in range(D // 16):
            zbuf[i, pl.ds(cc * 16, 16)] = jnp.zeros((16,), jnp.float32)
        return c
    lax.fori_loop(0, 16, fill_zero, 0)

    def zero_acc(kk, c):
        pltpu.sync_copy(zbuf, degacc.at[pl.ds(sid * RPT + kk * 16, 16)])
        return c
    lax.fori_loop(0, RPT // 16, zero_acc, 0)
    plsc.subcore_barrier()

    pltpu.sync_copy(e2_hbm.at[1, pl.ds(b0, NB + 7)], didx)

    def step(j, c):
        pltpu.sync_copy(ones_v, degacc.at[didx.at[off + j]], add=True)
        return c
    lax.fori_loop(0, NB, step, 0)

    plsc.subcore_barrier()
    pltpu.sync_copy(degacc.at[pl.ds(sid * RPT, RPT)],
                    degp_hbm.at[cid, pl.ds(sid * RPT, RPT)])


@functools.partial(
    pl.kernel,
    out_type=jax.ShapeDtypeStruct((NC, NPAD, D), jnp.float32),
    mesh=_mesh,
    scratch_types=[
        pltpu.VMEM((WIN, K), jnp.int32),
        pltpu.VMEM((WIN, K), jnp.int32),
        pltpu.VMEM((K, D), jnp.float32),
        pltpu.VMEM((K, D), jnp.float32),
        pltpu.VMEM((16, D), jnp.float32),
        pltpu.VMEM_SHARED((NPAD, D), jnp.float32),
        pltpu.SemaphoreType.DMA,
        pltpu.SemaphoreType.DMA,
    ],
)
def _sc_agg(h2_hbm, e2_hbm, accp_hbm, sidx, didx, ga, gb, zbuf,
            acc, sema, semb):
    cid = lax.axis_index("c")
    sid = lax.axis_index("s")
    nb_w = jnp.where(cid == 0, NB0, NB1)
    start = jnp.where(cid == 0, sid * NB0, NS * NB0 + sid * NB1)
    b0 = pl.multiple_of(start // 8 * 8, 8)
    off = start - b0

    def fill_zero(i, c):
        for cc in range(D // 16):
            zbuf[i, pl.ds(cc * 16, 16)] = jnp.zeros((16,), jnp.float32)
        return c
    lax.fori_loop(0, 16, fill_zero, 0)

    def zero_acc(kk, c):
        pltpu.sync_copy(zbuf, acc.at[pl.ds(sid * RPT + kk * 16, 16)])
        return c
    lax.fori_loop(0, RPT // 16, zero_acc, 0)
    plsc.subcore_barrier()

    bufs = (ga, gb)
    sems = (sema, semb)
    for c in range(NCH):
        trips = jnp.clip(nb_w - c * CH, 0, CH)
        pltpu.sync_copy(e2_hbm.at[0, pl.ds(b0 + c * CH, WIN)], sidx)
        pltpu.sync_copy(e2_hbm.at[1, pl.ds(b0 + c * CH, WIN)], didx)

        @pl.when(trips > 0)
        def _():
            pltpu.async_copy(h2_hbm.at[sidx.at[off]], ga, sema)

        @pl.when(trips > 1)
        def _():
            pltpu.async_copy(h2_hbm.at[sidx.at[off + 1]], gb, semb)

        def step(j, _, trips=trips):
            def lane(buf, sem):
                pltpu.make_async_copy(
                    h2_hbm.at[sidx.at[off + j]], buf, sem).wait()
                pltpu.sync_copy(buf, acc.at[didx.at[off + j]], add=True)

                @pl.when(j < trips - 2)
                def _():
                    pltpu.async_copy(
                        h2_hbm.at[sidx.at[off + j + 2]], buf, sem)

            @pl.when(j % 2 == 0)
            def _():
                lane(bufs[0], sems[0])

            @pl.when(j % 2 == 1)
            def _():
                lane(bufs[1], sems[1])
            return 0
        lax.fori_loop(0, trips, step, 0)

    plsc.subcore_barrier()
    pltpu.sync_copy(acc.at[pl.ds(sid * RPT, RPT)],
                    accp_hbm.at[cid, pl.ds(sid * RPT, RPT)])


def _dis_from_degp(degp):
    deg = (degp[0] + degp[1])[:N, 0:1]
    return jnp.where(deg > 0, lax.rsqrt(jnp.maximum(deg, 1e-12)), 0.0)


def _tc_mm_body(x_ref, w_ref, h_ref):
    h_ref[...] = jnp.dot(
        x_ref[...], w_ref[...], preferred_element_type=jnp.float32)


def _tc_pre_body(h_ref, degp_ref, h2_ref):
    dis = _dis_from_degp(degp_ref[...])
    h2_ref[...] = h_ref[...] * dis


def _bn_relu_part(accp, degp, b, g, be):
    dis = _dis_from_degp(degp)
    h = (accp[0, :N] + accp[1, :N]) * dis + b
    mean = jnp.mean(h, axis=0, keepdims=True)
    c = h - mean
    var = jnp.mean(c * c, axis=0, keepdims=True)
    hn = c * lax.rsqrt(var + EPS) * g + be
    return jnp.maximum(hn, 0.0), dis


def _tc_mid_body(accp_ref, degp_ref, b_ref, g_ref, be_ref, wn_ref, h2_ref):
    r, dis = _bn_relu_part(accp_ref[...], degp_ref[...], b_ref[...],
                           g_ref[...], be_ref[...])
    h2_ref[...] = jnp.dot(
        r, wn_ref[...], preferred_element_type=jnp.float32) * dis


def _tc_post_body(accp_ref, degp_ref, b_ref, g_ref, be_ref, out_ref):
    r, _ = _bn_relu_part(accp_ref[...], degp_ref[...], b_ref[...],
                         g_ref[...], be_ref[...])
    out_ref[...] = r


_tc_mm = pl.pallas_call(
    _tc_mm_body, out_shape=jax.ShapeDtypeStruct((N, D), jnp.float32))
_tc_pre = pl.pallas_call(
    _tc_pre_body, out_shape=jax.ShapeDtypeStruct((N, D), jnp.float32))
_tc_mid = pl.pallas_call(
    _tc_mid_body, out_shape=jax.ShapeDtypeStruct((N, D), jnp.float32))
_tc_post = pl.pallas_call(
    _tc_post_body, out_shape=jax.ShapeDtypeStruct((N, D), jnp.float32))


def kernel(x, edge_index, W0, b0, g0, be0, W1, b1, g1, be1, W2, b2, g2, be2):
    loop = jnp.arange(N, dtype=edge_index.dtype)
    pad = EPAD - ET
    tail = jnp.stack([jnp.concatenate([loop, jnp.zeros((pad,), jnp.int32)]),
                      jnp.concatenate([loop, jnp.full((pad,), N, jnp.int32)])])
    e2 = jnp.concatenate([edge_index, tail], axis=1).reshape(2, NBLK, K)

    h0 = _tc_mm(x, W0)
    degp = _sc_deg(e2)
    h2 = _tc_pre(h0, degp)
    params = [(b0, g0, be0, W1), (b1, g1, be1, W2), (b2, g2, be2, None)]
    for b, g, be, Wn in params:
        accp = _sc_agg(h2, e2)
        br, gr, ber = b.reshape(1, D), g.reshape(1, D), be.reshape(1, D)
        if Wn is not None:
            h2 = _tc_mid(accp, degp, br, gr, ber, Wn)
        else:
            h2 = _tc_post(accp, degp, br, gr, ber)
    return h2


# speedup vs baseline: 1.0273x; 1.0273x over previous
import functools

import jax
import jax.numpy as jnp
from jax import lax
from jax.experimental import pallas as pl
from jax.experimental.pallas import tpu as pltpu
from jax.experimental.pallas import tpu_sc as plsc

N = 10000
D = 128
E = 320000
ET = E + N
K = 128
NC = 2
NS = 16
NW = NC * NS
NB = -(-ET // (NW * K))
NB0 = 115
NB1 = 47
CH = 40
NCH = 3
WIN = CH + 8
_max_start = max(NB * (NW - 1), NS * NB0 + (NS - 1) * NB1)
NBLK = (_max_start // 8 * 8 + (NCH - 1) * CH + WIN + 7) // 8 * 8
EPAD = NBLK * K
NPAD = 10240
RPT = NPAD // NS
EPS = 1e-5

_mesh = plsc.VectorSubcoreMesh(
    core_axis_name="c", subcore_axis_name="s", num_cores=NC, num_subcores=NS)


@functools.partial(
    pl.kernel,
    out_type=jax.ShapeDtypeStruct((NC, NPAD, D), jnp.float32),
    mesh=_mesh,
    scratch_types=[
        pltpu.VMEM((NB + 7, K), jnp.int32),
        pltpu.VMEM((K, D), jnp.float32),
        pltpu.VMEM((16, D), jnp.float32),
        pltpu.VMEM_SHARED((NPAD, D), jnp.float32),
    ],
)
def _sc_deg(e2_hbm, degp_hbm, didx, ones_v, zbuf, degacc):
    cid = lax.axis_index("c")
    sid = lax.axis_index("s")
    w = cid * NS + sid
    b0 = pl.multiple_of(NB * w // 8 * 8, 8)
    off = NB * w - b0

    def fill_ones(i, c):
        for cc in range(D // 16):
            ones_v[i, pl.ds(cc * 16, 16)] = jnp.ones((16,), jnp.float32)
        return c
    lax.fori_loop(0, K, fill_ones, 0)

    def fill_zero(i, c):
        for cc in range(D // 16):
            zbuf[i, pl.ds(cc * 16, 16)] = jnp.zeros((16,), jnp.float32)
        return c
    lax.fori_loop(0, 16, fill_zero, 0)

    def zero_acc(kk, c):
        pltpu.sync_copy(zbuf, degacc.at[pl.ds(sid * RPT + kk * 16, 16)])
        return c
    lax.fori_loop(0, RPT // 16, zero_acc, 0)
    plsc.subcore_barrier()

    pltpu.sync_copy(e2_hbm.at[1, pl.ds(b0, NB + 7)], didx)

    def step(j, c):
        pltpu.sync_copy(ones_v, degacc.at[didx.at[off + j]], add=True)
        return c
    lax.fori_loop(0, NB, step, 0)

    plsc.subcore_barrier()
    pltpu.sync_copy(degacc.at[pl.ds(sid * RPT, RPT)],
                    degp_hbm.at[cid, pl.ds(sid * RPT, RPT)])


@functools.partial(
    pl.kernel,
    out_type=jax.ShapeDtypeStruct((NC, NPAD, D), jnp.float32),
    mesh=_mesh,
    scratch_types=[
        pltpu.VMEM((WIN, K), jnp.int32),
        pltpu.VMEM((WIN, K), jnp.int32),
        pltpu.VMEM((K, D), jnp.float32),
        pltpu.VMEM((K, D), jnp.float32),
        pltpu.VMEM((16, D), jnp.float32),
        pltpu.VMEM_SHARED((NPAD, D), jnp.float32),
        pltpu.SemaphoreType.DMA,
        pltpu.SemaphoreType.DMA,
    ],
)
def _sc_agg(h2_hbm, e2_hbm, accp_hbm, sidx, didx, ga, gb, zbuf,
            acc, sema, semb):
    cid = lax.axis_index("c")
    sid = lax.axis_index("s")
    nb_w = jnp.where(cid == 0, NB0, NB1)
    start = jnp.where(cid == 0, sid * NB0, NS * NB0 + sid * NB1)
    b0 = pl.multiple_of(start // 8 * 8, 8)
    off = start - b0

    def fill_zero(i, c):
        for cc in range(D // 16):
            zbuf[i, pl.ds(cc * 16, 16)] = jnp.zeros((16,), jnp.float32)
        return c
    lax.fori_loop(0, 16, fill_zero, 0)

    def zero_acc(kk, c):
        pltpu.sync_copy(zbuf, acc.at[pl.ds(sid * RPT + kk * 16, 16)])
        return c
    lax.fori_loop(0, RPT // 16, zero_acc, 0)
    plsc.subcore_barrier()

    bufs = (ga, gb)
    sems = (sema, semb)
    for c in range(NCH):
        trips = jnp.clip(nb_w - c * CH, 0, CH)
        pltpu.sync_copy(e2_hbm.at[0, pl.ds(b0 + c * CH, WIN)], sidx)
        pltpu.sync_copy(e2_hbm.at[1, pl.ds(b0 + c * CH, WIN)], didx)

        @pl.when(trips > 0)
        def _():
            pltpu.async_copy(h2_hbm.at[sidx.at[off]], ga, sema)

        @pl.when(trips > 1)
        def _():
            pltpu.async_copy(h2_hbm.at[sidx.at[off + 1]], gb, semb)

        def step(j, _, trips=trips):
            def lane(buf, sem):
                pltpu.make_async_copy(
                    h2_hbm.at[sidx.at[off + j]], buf, sem).wait()
                pltpu.sync_copy(buf, acc.at[didx.at[off + j]], add=True)

                @pl.when(j < trips - 2)
                def _():
                    pltpu.async_copy(
                        h2_hbm.at[sidx.at[off + j + 2]], buf, sem)

            @pl.when(j % 2 == 0)
            def _():
                lane(bufs[0], sems[0])

            @pl.when(j % 2 == 1)
            def _():
                lane(bufs[1], sems[1])
            return 0
        lax.fori_loop(0, trips, step, 0)

    plsc.subcore_barrier()
    pltpu.sync_copy(acc.at[pl.ds(sid * RPT, RPT)],
                    accp_hbm.at[cid, pl.ds(sid * RPT, RPT)])


def _dis_from_degp(degp):
    deg = (degp[0] + degp[1])[:N, 0:1]
    return jnp.where(deg > 0, lax.rsqrt(jnp.maximum(deg, 1e-12)), 0.0)


def _tc_pre_body(x_ref, w_ref, degp_ref, h2_ref):
    dis = _dis_from_degp(degp_ref[...])
    h = jnp.dot(x_ref[...], w_ref[...], preferred_element_type=jnp.float32)
    h2_ref[...] = h * dis


def _bn_relu_part(accp, degp, b, g, be):
    dis = _dis_from_degp(degp)
    h = (accp[0, :N] + accp[1, :N]) * dis + b
    mean = jnp.mean(h, axis=0, keepdims=True)
    c = h - mean
    var = jnp.mean(c * c, axis=0, keepdims=True)
    hn = c * lax.rsqrt(var + EPS) * g + be
    return jnp.maximum(hn, 0.0), dis


def _tc_mid_body(accp_ref, degp_ref, b_ref, g_ref, be_ref, wn_ref, h2_ref):
    r, dis = _bn_relu_part(accp_ref[...], degp_ref[...], b_ref[...],
                           g_ref[...], be_ref[...])
    h2_ref[...] = jnp.dot(
        r, wn_ref[...], preferred_element_type=jnp.float32) * dis


def _tc_post_body(accp_ref, degp_ref, b_ref, g_ref, be_ref, out_ref):
    r, _ = _bn_relu_part(accp_ref[...], degp_ref[...], b_ref[...],
                         g_ref[...], be_ref[...])
    out_ref[...] = r


_tc_pre = pl.pallas_call(
    _tc_pre_body, out_shape=jax.ShapeDtypeStruct((N, D), jnp.float32))
_tc_mid = pl.pallas_call(
    _tc_mid_body, out_shape=jax.ShapeDtypeStruct((N, D), jnp.float32))
_tc_post = pl.pallas_call(
    _tc_post_body, out_shape=jax.ShapeDtypeStruct((N, D), jnp.float32))


def kernel(x, edge_index, W0, b0, g0, be0, W1, b1, g1, be1, W2, b2, g2, be2):
    loop = jnp.arange(N, dtype=edge_index.dtype)
    pad = EPAD - ET
    tail = jnp.stack([jnp.concatenate([loop, jnp.zeros((pad,), jnp.int32)]),
                      jnp.concatenate([loop, jnp.full((pad,), N, jnp.int32)])])
    e2 = jnp.concatenate([edge_index, tail], axis=1).reshape(2, NBLK, K)

    degp = _sc_deg(e2)
    h2 = _tc_pre(x, W0, degp)
    params = [(b0, g0, be0, W1), (b1, g1, be1, W2), (b2, g2, be2, None)]
    for b, g, be, Wn in params:
        accp = _sc_agg(h2, e2)
        br, gr, ber = b.reshape(1, D), g.reshape(1, D), be.reshape(1, D)
        if Wn is not None:
            h2 = _tc_mid(accp, degp, br, gr, ber, Wn)
        else:
            h2 = _tc_post(accp, degp, br, gr, ber)
    return h2
